# baseline (speedup 1.0000x reference)
import functools

import jax
import jax.numpy as jnp
from jax import lax
from jax.experimental import pallas as pl
from jax.experimental.pallas import tpu as pltpu
from jax.experimental.pallas import tpu_sc as plsc

N = 160000
D = 512
T = 12
TP = 16
S = 4096
L = 16
NF = D // L

NC = 2
NS = 16
NW = NC * NS
SEG_PER_W = S // NW
OFF_SLICE = 144
CH = 64
CH8 = CH + 8
RB = 4


_TW_BLK = 2000


def _tw_body(x_ref, w_ref, b_ref, o_ref):
    z = jnp.dot(x_ref[...], w_ref[...], preferred_element_type=jnp.float32)
    o_ref[...] = jax.nn.sigmoid(z + b_ref[...])


def _task_weights(feats, ws_t_pad, bs_pad):
    return pl.pallas_call(
        _tw_body,
        grid=(N // _TW_BLK,),
        in_specs=[
            pl.BlockSpec((_TW_BLK, D), lambda i: (i, 0)),
            pl.BlockSpec((D, TP), lambda i: (0, 0)),
            pl.BlockSpec((1, TP), lambda i: (0, 0)),
        ],
        out_specs=pl.BlockSpec((_TW_BLK, TP), lambda i: (i, 0)),
        out_shape=jax.ShapeDtypeStruct((N, TP), jnp.float32),
    )(feats, ws_t_pad, bs_pad)


_MESH = plsc.VectorSubcoreMesh(core_axis_name="c", subcore_axis_name="s")


@functools.partial(
    pl.kernel,
    out_type=jax.ShapeDtypeStruct((T * S * D,), jnp.float32),
    mesh=_MESH,
    scratch_types=[
        pltpu.VMEM((OFF_SLICE,), jnp.int32),
        pltpu.VMEM((CH8, D), jnp.float32),
        pltpu.VMEM((CH8 * TP,), jnp.float32),
        pltpu.VMEM((T * D,), jnp.float32),
        pltpu.SemaphoreType.DMA,
    ],
)
def _sc_seg_sum(feats_hbm, tw_hbm, off_hbm, out_hbm, off_v, fbuf, wbuf, acc,
                sem):
    cid = lax.axis_index("c")
    sid = lax.axis_index("s")
    wid = sid * NC + cid
    s0 = wid * SEG_PER_W

    pltpu.sync_copy(off_hbm.at[pl.ds(s0, OFF_SLICE)], off_v)

    zero16 = jnp.zeros((L,), jnp.float32)

    def zero_frag(f, _):
        for t in range(T):
            acc[pl.ds(t * D + f * L, L)] = zero16
        return 0

    def seg_body(sl, _):
        ovec = off_v[pl.ds(sl, L)]
        r0 = ovec[0]
        r1 = ovec[1]
        n = r1 - r0
        lax.fori_loop(0, NF, zero_frag, 0, unroll=False)

        def chunk_body(k, _):
            start = r0 + k * CH
            cs = jnp.minimum((start // 8) * 8, N - CH8)
            dlt = start - cs
            pltpu.sync_copy(feats_hbm.at[pl.ds(cs, CH8)], fbuf)
            pltpu.sync_copy(tw_hbm.at[pl.ds(cs * TP, CH8 * TP)], wbuf)
            m = jnp.minimum(n - k * CH, CH)
            nb = m // RB

            def batch_body(b, _):
                ri0 = dlt + b * RB
                wv = [wbuf[pl.ds((ri0 + r) * TP, TP)] for r in range(RB)]
                w = [[wv[r][t] for t in range(T)] for r in range(RB)]

                def frag_body(f, _):
                    v = [fbuf[ri0 + r, pl.ds(f * L, L)] for r in range(RB)]
                    for t in range(T):
                        p = w[0][t] * v[0]
                        for r in range(1, RB):
                            p = p + w[r][t] * v[r]
                        plsc.addupdate(acc.at[pl.ds(t * D + f * L, L)], p)
                    return 0

                lax.fori_loop(0, NF, frag_body, 0, unroll=2)
                return 0

            lax.fori_loop(0, nb, batch_body, 0, unroll=False)

            def row_body(i, _):
                ri = dlt + i
                wvec = wbuf[pl.ds(ri * TP, TP)]
                w = [wvec[t] for t in range(T)]

                def frag_body(f, _):
                    v = fbuf[ri, pl.ds(f * L, L)]
                    for t in range(T):
                        plsc.addupdate(acc.at[pl.ds(t * D + f * L, L)],
                                       w[t] * v)
                    return 0

                lax.fori_loop(0, NF, frag_body, 0, unroll=2)
                return 0

            lax.fori_loop(nb * RB, m, row_body, 0, unroll=False)
            return 0

        nch = (n + CH - 1) // CH
        lax.fori_loop(0, nch, chunk_body, 0, unroll=False)
        s = s0 + sl
        copies = [pltpu.async_copy(acc.at[pl.ds(t * D, D)],
                                   out_hbm.at[pl.ds((t * S + s) * D, D)],
                                   sem)
                  for t in range(T)]
        for c in copies:
            c.wait()
        return 0

    lax.fori_loop(0, SEG_PER_W, seg_body, 0, unroll=False)


def kernel(feats, segment_ids, Ws, bs, shared_W, shared_b):
    ws_t_pad = jnp.zeros((D, TP), jnp.float32).at[:, :T].set(Ws.T)
    bs_pad = jnp.zeros((1, TP), jnp.float32).at[0, :T].set(bs)
    tw = _task_weights(feats, ws_t_pad, bs_pad)

    offs = jnp.searchsorted(segment_ids, jnp.arange(S + 1, dtype=jnp.int32),
                            side="left").astype(jnp.int32)
    offs = jnp.concatenate(
        [offs, jnp.full(((NW - 1) * SEG_PER_W + OFF_SLICE) - (S + 1), N,
                        dtype=jnp.int32)])

    out = _sc_seg_sum(feats, tw.reshape(-1), offs)
    return out.reshape(T, S, D)


# speedup vs baseline: 2.2362x; 2.2362x over previous
import functools

import jax
import jax.numpy as jnp
from jax import lax
from jax.experimental import pallas as pl
from jax.experimental.pallas import tpu as pltpu
from jax.experimental.pallas import tpu_sc as plsc

N = 160000
D = 512
T = 12
TP = 16
S = 4096
L = 16
NF = D // L

NC = 2
NS = 16
NW = NC * NS
SEG_PER_W = S // NW
OFF_SLICE = 144
CH = 64
CH8 = CH + 8
RB = 4


_TW_BLK = 2000


def _tw_body(x_ref, w_ref, b_ref, o_ref):
    z = jnp.dot(x_ref[...], w_ref[...], preferred_element_type=jnp.float32)
    o_ref[...] = jax.nn.sigmoid(z + b_ref[...])


def _task_weights(feats, ws_t_pad, bs_pad):
    return pl.pallas_call(
        _tw_body,
        grid=(N // _TW_BLK,),
        in_specs=[
            pl.BlockSpec((_TW_BLK, D), lambda i: (i, 0)),
            pl.BlockSpec((D, TP), lambda i: (0, 0)),
            pl.BlockSpec((1, TP), lambda i: (0, 0)),
        ],
        out_specs=pl.BlockSpec((_TW_BLK, TP), lambda i: (i, 0)),
        out_shape=jax.ShapeDtypeStruct((N, TP), jnp.float32),
    )(feats, ws_t_pad, bs_pad)


_MESH = plsc.VectorSubcoreMesh(core_axis_name="c", subcore_axis_name="s")


@functools.partial(
    pl.kernel,
    out_type=jax.ShapeDtypeStruct((T * S * D,), jnp.float32),
    mesh=_MESH,
    scratch_types=[
        pltpu.VMEM((OFF_SLICE,), jnp.int32),
        pltpu.VMEM((CH8, D), jnp.float32),
        pltpu.VMEM((CH8 * TP,), jnp.float32),
        pltpu.VMEM((T * D,), jnp.float32),
        pltpu.SemaphoreType.DMA,
    ],
)
def _sc_seg_sum(feats_hbm, tw_hbm, off_hbm, out_hbm, off_v, fbuf, wbuf, acc,
                sem):
    cid = lax.axis_index("c")
    sid = lax.axis_index("s")
    wid = sid * NC + cid
    s0 = wid * SEG_PER_W

    pltpu.sync_copy(off_hbm.at[pl.ds(s0, OFF_SLICE)], off_v)

    zero16 = jnp.zeros((L,), jnp.float32)

    def zero_frag(f, _):
        for t in range(T):
            acc[pl.ds(t * D + f * L, L)] = zero16
        return 0

    def seg_body(sl, _):
        ovec = off_v[pl.ds(sl, L)]
        r0 = ovec[0]
        r1 = ovec[1]
        n = r1 - r0
        lax.fori_loop(0, NF, zero_frag, 0, unroll=False)

        def chunk_body(k, _):
            start = r0 + k * CH
            cs = jnp.minimum((start // 8) * 8, N - CH8)
            dlt = start - cs
            pltpu.sync_copy(feats_hbm.at[pl.ds(cs, CH8)], fbuf)
            pltpu.sync_copy(tw_hbm.at[pl.ds(cs * TP, CH8 * TP)], wbuf)
            m = jnp.minimum(n - k * CH, CH)
            nb = m // RB

            def batch_body(b, _):
                ri0 = dlt + b * RB
                wv = [wbuf[pl.ds((ri0 + r) * TP, TP)] for r in range(RB)]
                w = [[wv[r][t] for t in range(T)] for r in range(RB)]

                def frag_body(f, _):
                    v = [fbuf[ri0 + r, pl.ds(f * L, L)] for r in range(RB)]
                    for t in range(T):
                        p = w[0][t] * v[0]
                        for r in range(1, RB):
                            p = p + w[r][t] * v[r]
                        plsc.addupdate(acc.at[pl.ds(t * D + f * L, L)], p)
                    return 0

                lax.fori_loop(0, NF, frag_body, 0, unroll=2)
                return 0

            lax.fori_loop(0, nb, batch_body, 0, unroll=False)

            def row_body(i, _):
                ri = dlt + i
                wvec = wbuf[pl.ds(ri * TP, TP)]
                w = [wvec[t] for t in range(T)]

                def frag_body(f, _):
                    v = fbuf[ri, pl.ds(f * L, L)]
                    for t in range(T):
                        plsc.addupdate(acc.at[pl.ds(t * D + f * L, L)],
                                       w[t] * v)
                    return 0

                lax.fori_loop(0, NF, frag_body, 0, unroll=2)
                return 0

            lax.fori_loop(nb * RB, m, row_body, 0, unroll=False)
            return 0

        nch = (n + CH - 1) // CH
        lax.fori_loop(0, nch, chunk_body, 0, unroll=False)
        s = s0 + sl
        copies = [pltpu.async_copy(acc.at[pl.ds(t * D, D)],
                                   out_hbm.at[pl.ds((t * S + s) * D, D)],
                                   sem)
                  for t in range(T)]
        for c in copies:
            c.wait()
        return 0

    lax.fori_loop(0, SEG_PER_W, seg_body, 0, unroll=False)


def kernel(feats, segment_ids, Ws, bs, shared_W, shared_b):
    ws_t_pad = jnp.zeros((D, TP), jnp.float32).at[:, :T].set(Ws.T)
    bs_pad = jnp.zeros((1, TP), jnp.float32).at[0, :T].set(bs)
    tw = _task_weights(feats, ws_t_pad, bs_pad)

    B2 = 64
    blocks = segment_ids.reshape(N // B2, B2)
    bmax = blocks.max(axis=1)
    svals = jnp.arange(S + 1, dtype=jnp.int32)
    jb = jnp.sum((bmax[None, :] < svals[:, None]).astype(jnp.int32),
                 axis=1)
    blk = blocks[jnp.clip(jb, 0, N // B2 - 1)]
    inner = jnp.sum((blk < svals[:, None]).astype(jnp.int32), axis=1)
    offs = jnp.minimum(jb * B2 + inner, N).astype(jnp.int32)
    offs = jnp.concatenate(
        [offs, jnp.full(((NW - 1) * SEG_PER_W + OFF_SLICE) - (S + 1), N,
                        dtype=jnp.int32)])

    out = _sc_seg_sum(feats, tw.reshape(-1), offs)
    return out.reshape(T, S, D)


# speedup vs baseline: 2.7302x; 1.2209x over previous
import functools

import jax
import jax.numpy as jnp
from jax import lax
from jax.experimental import pallas as pl
from jax.experimental.pallas import tpu as pltpu
from jax.experimental.pallas import tpu_sc as plsc

N = 160000
D = 512
T = 12
TP = 16
S = 4096
L = 16
NF = D // L

NC = 2
NS = 16
NW = NC * NS
SEG_PER_W = S // NW
OFF_SLICE = 144
CH = 64
CH8 = CH + 8
RB = 4


_TW_BLK = 2000


def _tw_body(x_ref, w_ref, b_ref, o_ref):
    z = jnp.dot(x_ref[...], w_ref[...], preferred_element_type=jnp.float32)
    o_ref[...] = jax.nn.sigmoid(z + b_ref[...])


def _task_weights(feats, ws_t_pad, bs_pad):
    return pl.pallas_call(
        _tw_body,
        grid=(N // _TW_BLK,),
        in_specs=[
            pl.BlockSpec((_TW_BLK, D), lambda i: (i, 0)),
            pl.BlockSpec((D, TP), lambda i: (0, 0)),
            pl.BlockSpec((1, TP), lambda i: (0, 0)),
        ],
        out_specs=pl.BlockSpec((_TW_BLK, TP), lambda i: (i, 0)),
        out_shape=jax.ShapeDtypeStruct((N, TP), jnp.float32),
    )(feats, ws_t_pad, bs_pad)


_MESH = plsc.VectorSubcoreMesh(core_axis_name="c", subcore_axis_name="s")


@functools.partial(
    pl.kernel,
    out_type=jax.ShapeDtypeStruct((T * S * D,), jnp.float32),
    mesh=_MESH,
    scratch_types=[
        pltpu.VMEM((OFF_SLICE,), jnp.int32),
        pltpu.VMEM((2, CH8, D), jnp.float32),
        pltpu.VMEM((2, CH8 * TP), jnp.float32),
        pltpu.VMEM((T * D,), jnp.float32),
        pltpu.SemaphoreType.DMA,
        pltpu.SemaphoreType.DMA,
        pltpu.SemaphoreType.DMA,
    ],
)
def _sc_seg_sum(feats_hbm, tw_hbm, off_hbm, out_hbm, off_v, fbuf, wbuf,
                acc, fsem0, fsem1, osem):
    cid = lax.axis_index("c")
    sid = lax.axis_index("s")
    wid = sid * NC + cid
    s0 = wid * SEG_PER_W

    pltpu.sync_copy(off_hbm.at[pl.ds(s0, OFF_SLICE)], off_v)
    fsems = (fsem0, fsem1)

    zero16 = jnp.zeros((L,), jnp.float32)

    def nseg(sl):
        ov = off_v[pl.ds(sl, L)]
        return ov[0], ov[1] - ov[0]

    def chunk_base(r):
        cs = jnp.minimum((r // 8) * 8, N - CH8)
        return cs, r - cs

    def prefetch(sl_next, b):
        r0, _n = nseg(sl_next)
        cs, _d = chunk_base(r0)
        pltpu.async_copy(feats_hbm.at[pl.ds(cs, CH8)], fbuf.at[b],
                         fsems[b])
        pltpu.async_copy(tw_hbm.at[pl.ds(cs * TP, CH8 * TP)],
                         wbuf.at[b], fsems[b])

    def drain_feats(b):
        pltpu.make_async_copy(feats_hbm.at[pl.ds(0, CH8)], fbuf.at[b],
                              fsems[b]).wait()
        pltpu.make_async_copy(tw_hbm.at[pl.ds(0, CH8 * TP)], wbuf.at[b],
                              fsems[b]).wait()

    def process_rows(b, acc, base, cnt):
        nb = cnt // RB

        def batch_body(bb, _):
            ri0 = base + bb * RB
            wv = [wbuf[b, pl.ds((ri0 + r) * TP, TP)] for r in range(RB)]
            w = [[wv[r][t] for t in range(T)] for r in range(RB)]

            def frag_body(f, _):
                v = [fbuf[b, ri0 + r, pl.ds(f * L, L)] for r in range(RB)]
                for t in range(T):
                    p = w[0][t] * v[0]
                    for r in range(1, RB):
                        p = p + w[r][t] * v[r]
                    plsc.addupdate(acc.at[pl.ds(t * D + f * L, L)], p)
                return 0

            lax.fori_loop(0, NF, frag_body, 0, unroll=2)
            return 0

        lax.fori_loop(0, nb, batch_body, 0, unroll=False)

        def row_body(i, _):
            ri = base + i
            wvec = wbuf[b, pl.ds(ri * TP, TP)]
            w = [wvec[t] for t in range(T)]

            def frag_body(f, _):
                v = fbuf[b, ri, pl.ds(f * L, L)]
                for t in range(T):
                    plsc.addupdate(acc.at[pl.ds(t * D + f * L, L)],
                                   w[t] * v)
                return 0

            lax.fori_loop(0, NF, frag_body, 0, unroll=2)
            return 0

        lax.fori_loop(nb * RB, cnt, row_body, 0, unroll=False)

    def phase(sl, b):
        r0, n = nseg(sl)
        prefetch(sl + 1, 1 - b)
        drain_feats(b)

        def zero_frag(f, _):
            for t in range(T):
                acc[pl.ds(t * D + f * L, L)] = zero16
            return 0

        lax.fori_loop(0, NF, zero_frag, 0, unroll=False)

        @pl.when(n > 0)
        def _():
            cs, dlt = chunk_base(r0)
            m0 = jnp.minimum(n, CH8 - dlt)
            process_rows(b, acc, dlt, m0)
            nex = (n - m0 + CH - 1) // CH

            def ov_body(c, _):
                st = r0 + m0 + c * CH
                cs2, dlt2 = chunk_base(st)
                pltpu.sync_copy(feats_hbm.at[pl.ds(cs2, CH8)], fbuf.at[b])
                pltpu.sync_copy(tw_hbm.at[pl.ds(cs2 * TP, CH8 * TP)],
                                wbuf.at[b])
                mc = jnp.minimum(n - m0 - c * CH, CH)
                process_rows(b, acc, dlt2, mc)
                return 0

            lax.fori_loop(0, nex, ov_body, 0, unroll=False)

        s = s0 + sl
        copies = [pltpu.async_copy(acc.at[pl.ds(t * D, D)],
                                   out_hbm.at[pl.ds((t * S + s) * D, D)],
                                   osem)
                  for t in range(T)]
        for c in copies:
            c.wait()

    prefetch(0, 0)

    def pair(kk, _):
        phase(2 * kk, 0)
        phase(2 * kk + 1, 1)
        return 0

    lax.fori_loop(0, SEG_PER_W // 2, pair, 0, unroll=False)
    drain_feats(0)


def kernel(feats, segment_ids, Ws, bs, shared_W, shared_b):
    ws_t_pad = jnp.zeros((D, TP), jnp.float32).at[:, :T].set(Ws.T)
    bs_pad = jnp.zeros((1, TP), jnp.float32).at[0, :T].set(bs)
    tw = _task_weights(feats, ws_t_pad, bs_pad)

    B2 = 64
    blocks = segment_ids.reshape(N // B2, B2)
    bmax = blocks.max(axis=1)
    svals = jnp.arange(S + 1, dtype=jnp.int32)
    jb = jnp.sum((bmax[None, :] < svals[:, None]).astype(jnp.int32),
                 axis=1)
    blk = blocks[jnp.clip(jb, 0, N // B2 - 1)]
    inner = jnp.sum((blk < svals[:, None]).astype(jnp.int32), axis=1)
    offs = jnp.minimum(jb * B2 + inner, N).astype(jnp.int32)
    offs = jnp.concatenate(
        [offs, jnp.full(((NW - 1) * SEG_PER_W + OFF_SLICE) - (S + 1), N,
                        dtype=jnp.int32)])

    out = _sc_seg_sum(feats, tw.reshape(-1), offs)
    return out.reshape(T, S, D)


# speedup vs baseline: 2.9031x; 1.0633x over previous
import functools

import jax
import jax.numpy as jnp
from jax import lax
from jax.experimental import pallas as pl
from jax.experimental.pallas import tpu as pltpu
from jax.experimental.pallas import tpu_sc as plsc

N = 160000
D = 512
T = 12
TP = 16
S = 4096
L = 16
NF = D // L

NC = 2
NS = 16
NW = NC * NS
SEG_PER_W = S // NW
OFF_SLICE = 144
CH = 64
CH8 = CH + 8
RB = 4


_TW_BLK = 2000


def _tw_body(x_ref, w_ref, b_ref, o_ref):
    z = jnp.dot(x_ref[...], w_ref[...], preferred_element_type=jnp.float32)
    o_ref[...] = jax.nn.sigmoid(z + b_ref[...])


def _task_weights(feats, ws_t_pad, bs_pad):
    return pl.pallas_call(
        _tw_body,
        grid=(N // _TW_BLK,),
        in_specs=[
            pl.BlockSpec((_TW_BLK, D), lambda i: (i, 0)),
            pl.BlockSpec((D, TP), lambda i: (0, 0)),
            pl.BlockSpec((1, TP), lambda i: (0, 0)),
        ],
        out_specs=pl.BlockSpec((_TW_BLK, TP), lambda i: (i, 0)),
        out_shape=jax.ShapeDtypeStruct((N, TP), jnp.float32),
    )(feats, ws_t_pad, bs_pad)


_MESH = plsc.VectorSubcoreMesh(core_axis_name="c", subcore_axis_name="s")


@functools.partial(
    pl.kernel,
    out_type=jax.ShapeDtypeStruct((T * S * D,), jnp.float32),
    mesh=_MESH,
    scratch_types=[
        pltpu.VMEM((OFF_SLICE,), jnp.int32),
        pltpu.VMEM((2, CH8, D), jnp.float32),
        pltpu.VMEM((2, CH8, TP), jnp.float32),
        pltpu.VMEM((T * D,), jnp.float32),
        pltpu.SemaphoreType.DMA,
        pltpu.SemaphoreType.DMA,
        pltpu.SemaphoreType.DMA,
    ],
)
def _sc_seg_sum(feats_hbm, tw_hbm, off_hbm, out_hbm, off_v, fbuf, wbuf,
                acc, fsem0, fsem1, osem):
    cid = lax.axis_index("c")
    sid = lax.axis_index("s")
    wid = sid * NC + cid
    s0 = wid * SEG_PER_W

    pltpu.sync_copy(off_hbm.at[pl.ds(s0, OFF_SLICE)], off_v)
    fsems = (fsem0, fsem1)

    zero16 = jnp.zeros((L,), jnp.float32)

    def nseg(sl):
        ov = off_v[pl.ds(sl, L)]
        return ov[0], ov[1] - ov[0]

    def chunk_base(r):
        cs = jnp.minimum((r // 8) * 8, N - CH8)
        return cs, r - cs

    def prefetch(sl_next, b):
        r0, _n = nseg(sl_next)
        cs, _d = chunk_base(r0)
        pltpu.async_copy(feats_hbm.at[pl.ds(cs, CH8)], fbuf.at[b],
                         fsems[b])
        pltpu.async_copy(tw_hbm.at[pl.ds(cs, CH8)], wbuf.at[b], fsems[b])

    def drain_feats(b):
        pltpu.make_async_copy(feats_hbm.at[pl.ds(0, CH8)], fbuf.at[b],
                              fsems[b]).wait()
        pltpu.make_async_copy(tw_hbm.at[pl.ds(0, CH8)], wbuf.at[b],
                              fsems[b]).wait()

    def process_rows(b, acc, base, cnt):
        nb = cnt // RB

        def batch_body(bb, _):
            ri0 = base + bb * RB
            wv = [wbuf[b, ri0 + r, pl.ds(0, TP)] for r in range(RB)]
            w = [[wv[r][t] for t in range(T)] for r in range(RB)]

            def frag_body(f, _):
                v = [fbuf[b, ri0 + r, pl.ds(f * L, L)] for r in range(RB)]
                for t in range(T):
                    p = w[0][t] * v[0]
                    for r in range(1, RB):
                        p = p + w[r][t] * v[r]
                    plsc.addupdate(acc.at[pl.ds(t * D + f * L, L)], p)
                return 0

            lax.fori_loop(0, NF, frag_body, 0, unroll=2)
            return 0

        lax.fori_loop(0, nb, batch_body, 0, unroll=False)

        def row_body(i, _):
            ri = base + i
            wvec = wbuf[b, ri, pl.ds(0, TP)]
            w = [wvec[t] for t in range(T)]

            def frag_body(f, _):
                v = fbuf[b, ri, pl.ds(f * L, L)]
                for t in range(T):
                    plsc.addupdate(acc.at[pl.ds(t * D + f * L, L)],
                                   w[t] * v)
                return 0

            lax.fori_loop(0, NF, frag_body, 0, unroll=2)
            return 0

        lax.fori_loop(nb * RB, cnt, row_body, 0, unroll=False)

    def phase(sl, b):
        r0, n = nseg(sl)
        prefetch(sl + 1, 1 - b)
        drain_feats(b)

        def zero_frag(f, _):
            for t in range(T):
                acc[pl.ds(t * D + f * L, L)] = zero16
            return 0

        lax.fori_loop(0, NF, zero_frag, 0, unroll=False)

        @pl.when(n > 0)
        def _():
            cs, dlt = chunk_base(r0)
            m0 = jnp.minimum(n, CH8 - dlt)
            process_rows(b, acc, dlt, m0)
            nex = (n - m0 + CH - 1) // CH

            def ov_body(c, _):
                st = r0 + m0 + c * CH
                cs2, dlt2 = chunk_base(st)
                pltpu.sync_copy(feats_hbm.at[pl.ds(cs2, CH8)], fbuf.at[b])
                pltpu.sync_copy(tw_hbm.at[pl.ds(cs2, CH8)], wbuf.at[b])
                mc = jnp.minimum(n - m0 - c * CH, CH)
                process_rows(b, acc, dlt2, mc)
                return 0

            lax.fori_loop(0, nex, ov_body, 0, unroll=False)

        s = s0 + sl
        copies = [pltpu.async_copy(acc.at[pl.ds(t * D, D)],
                                   out_hbm.at[pl.ds((t * S + s) * D, D)],
                                   osem)
                  for t in range(T)]
        for c in copies:
            c.wait()

    prefetch(0, 0)

    def pair(kk, _):
        phase(2 * kk, 0)
        phase(2 * kk + 1, 1)
        return 0

    lax.fori_loop(0, SEG_PER_W // 2, pair, 0, unroll=False)
    drain_feats(0)


def kernel(feats, segment_ids, Ws, bs, shared_W, shared_b):
    ws_t_pad = jnp.zeros((D, TP), jnp.float32).at[:, :T].set(Ws.T)
    bs_pad = jnp.zeros((1, TP), jnp.float32).at[0, :T].set(bs)
    tw = _task_weights(feats, ws_t_pad, bs_pad)

    B2 = 64
    blocks = segment_ids.reshape(N // B2, B2)
    bmax = blocks.max(axis=1)
    svals = jnp.arange(S + 1, dtype=jnp.int32)
    jb = jnp.sum((bmax[None, :] < svals[:, None]).astype(jnp.int32),
                 axis=1)
    blk = blocks[jnp.clip(jb, 0, N // B2 - 1)]
    inner = jnp.sum((blk < svals[:, None]).astype(jnp.int32), axis=1)
    offs = jnp.minimum(jb * B2 + inner, N).astype(jnp.int32)
    offs = jnp.concatenate(
        [offs, jnp.full(((NW - 1) * SEG_PER_W + OFF_SLICE) - (S + 1), N,
                        dtype=jnp.int32)])

    out = _sc_seg_sum(feats, tw, offs)
    return out.reshape(T, S, D)
